# baseline (speedup 1.0000x reference)
import functools
import jax
import jax.numpy as jnp
from jax import lax
from jax.experimental import pallas as pl
from jax.experimental.pallas import tpu as pltpu
from jax.experimental.pallas import tpu_sc as plsc

N = 10000
NP = 10240
NSLAB = 640
E = 320000
CH = 128
EPW = 10240
NCHUNK = EPW // CH
EP = EPW * 32
D = 128
G = 64
NC, NS, L = 2, 16, 16

_mesh = lambda: plsc.VectorSubcoreMesh(
    core_axis_name="c", subcore_axis_name="s", num_cores=NC, num_subcores=NS)


def _zero_vmem_rows(buf, nrows):
  z = jnp.zeros((L,), jnp.float32)
  for r in range(nrows):
    for q in range(D // L):
      buf[r, pl.ds(q * L, L)] = z


def _deg_body(dstp, degb, accd, didx, ones_v, zslab, dslab, rowbuf):
  c = lax.axis_index("c")
  s = lax.axis_index("s")

  @pl.when(c == 0)
  def _():
    z = jnp.zeros((L,), jnp.float32)
    def zb(i, _):
      for q in range(D // L):
        zslab[pl.ds(i * D + q * L, L)] = z
      return 0
    lax.fori_loop(0, NSLAB // D, zb, 0)
    pltpu.sync_copy(zslab, accd.at[pl.ds(s * NSLAB, NSLAB)])
    one = jnp.ones((L,), jnp.float32)
    for q in range(CH // L):
      ones_v[pl.ds(q * L, L)] = one
    plsc.subcore_barrier()

    nch = EP // NS // CH
    def chunk(k, _):
      base = s * (EP // NS) + k * CH
      pltpu.sync_copy(dstp.at[pl.ds(base, CH)], didx)
      pltpu.sync_copy(ones_v, accd.at[didx], add=True)
      return 0
    lax.fori_loop(0, nch, chunk, 0)
    plsc.subcore_barrier()

    pltpu.sync_copy(accd.at[pl.ds(s * NSLAB, NSLAB)], dslab)
    def wb(b, _):
      dvec = dslab[pl.ds(b * L, L)]
      for r in range(L):
        dval = dvec[r]
        v = jnp.full((L,), dval, jnp.float32)
        for q in range(D // L):
          rowbuf[r, pl.ds(q * L, L)] = v
      pltpu.sync_copy(rowbuf, degb.at[pl.ds(s * NSLAB + b * L, L)])
      return 0
    lax.fori_loop(0, NSLAB // L, wb, 0)


def _deg_sc(dstp):
  return pl.kernel(
      _deg_body,
      out_type=jax.ShapeDtypeStruct((NP, D), jnp.float32),
      mesh=_mesh(),
      scratch_types=[
          pltpu.VMEM_SHARED((NP,), jnp.float32),
          pltpu.VMEM((CH,), jnp.int32),
          pltpu.VMEM((CH,), jnp.float32),
          pltpu.VMEM((NSLAB,), jnp.float32),
          pltpu.VMEM((NSLAB,), jnp.float32),
          pltpu.VMEM((L, D), jnp.float32),
      ],
  )(dstp)


def _agg_body(hp, srcp, dst3, out, acc, sidx0, sidx1, didx, rows0, rows1,
              sem0, sem1):
  c = lax.axis_index("c")
  s = lax.axis_index("s")
  wid = c * NS + s
  ebase = wid * EPW

  pltpu.sync_copy(dst3.at[wid], didx)

  _zero_vmem_rows(rows0, CH)
  def zb(i, _):
    pltpu.sync_copy(rows0, acc.at[pl.ds(s * NSLAB + i * CH, CH)])
    return 0
  lax.fori_loop(0, NSLAB // CH, zb, 0)
  plsc.subcore_barrier()

  def fetch(k, sidx, buf, sem):
    pltpu.sync_copy(srcp.at[pl.ds(ebase + k * CH, CH)], sidx)
    pltpu.async_copy(hp.at[sidx], buf, sem)
  fetch(0, sidx0, rows0, sem0)
  fetch(1, sidx1, rows1, sem1)
  def pair(j, _):
    k = j * 2
    pltpu.make_async_copy(hp.at[sidx0], rows0, sem0).wait()
    pltpu.sync_copy(rows0, acc.at[didx.at[k]], add=True)

    @pl.when(j < NCHUNK // 2 - 1)
    def _():
      fetch(k + 2, sidx0, rows0, sem0)
    pltpu.make_async_copy(hp.at[sidx1], rows1, sem1).wait()
    pltpu.sync_copy(rows1, acc.at[didx.at[k + 1]], add=True)

    @pl.when(j < NCHUNK // 2 - 1)
    def _():
      fetch(k + 3, sidx1, rows1, sem1)
    return 0
  lax.fori_loop(0, NCHUNK // 2, pair, 0)
  plsc.subcore_barrier()

  pltpu.sync_copy(acc.at[pl.ds(s * NSLAB, NSLAB)],
                  out.at[c, pl.ds(s * NSLAB, NSLAB)])


def _agg_sc(hp, srcp, dst3):
  return pl.kernel(
      _agg_body,
      out_type=jax.ShapeDtypeStruct((NC, NP, D), jnp.float32),
      mesh=_mesh(),
      scratch_types=[
          pltpu.VMEM_SHARED((NP, D), jnp.float32),
          pltpu.VMEM((CH,), jnp.int32),
          pltpu.VMEM((CH,), jnp.int32),
          pltpu.VMEM((NCHUNK, CH), jnp.int32),
          pltpu.VMEM((CH, D), jnp.float32),
          pltpu.VMEM((CH, D), jnp.float32),
          pltpu.SemaphoreType.DMA,
          pltpu.SemaphoreType.DMA,
      ],
  )(hp, srcp, dst3)


def _pool_body(y, batch, means, maxs, bbuf, rbuf, omean, omax):
  c = lax.axis_index("c")
  s = lax.axis_index("s")
  w = c * NS + s

  pltpu.sync_copy(batch, bbuf)

  def lower_bound(gval):
    one = jnp.ones((L,), jnp.int32)
    zero = jnp.zeros((L,), jnp.int32)
    def sb(i, acc):
      v = bbuf[pl.ds(i * L, L)]
      return acc + jnp.minimum(jnp.maximum(gval - v, zero), one)
    acc = lax.fori_loop(0, N // L, sb, jnp.zeros((L,), jnp.int32))
    tot = acc[0]
    for j in range(1, L):
      tot = tot + acc[j]
    return tot

  g0 = w * 2
  lb0 = lower_bound(g0)
  lb1 = lower_bound(g0 + 1)
  lb2 = lower_bound(g0 + 2)

  NV = D // L
  BIG = jnp.float32(3.0e38)
  neg_big = jnp.full((L,), -BIG, jnp.float32)
  zeros = jnp.zeros((L,), jnp.float32)

  for gi, (start, end) in enumerate([(lb0, lb1), (lb1, lb2)]):
    astart = (start // L) * L
    nch = (end - astart + L - 1) // L
    def chunk(k, carry):
      sums, mxs = carry
      base = astart + k * L
      pltpu.sync_copy(y.at[pl.ds(base, L)], rbuf)
      for r in range(L):
        rg = base + r
        valid = jnp.logical_and(rg >= start, rg < end)
        vf = jnp.where(valid, jnp.float32(1.0), jnp.float32(0.0))
        vm = jnp.full((L,), vf, jnp.float32)
        vbias = jnp.full((L,), (vf - 1.0) * BIG, jnp.float32)
        for q in range(NV):
          v = rbuf[r, pl.ds(q * L, L)]
          sums = sums[:q] + (sums[q] + v * vm,) + sums[q + 1:]
          mxs = mxs[:q] + (jnp.maximum(mxs[q], v * vm + vbias),) + mxs[q + 1:]
      return (sums, mxs)
    init = (tuple(zeros for _ in range(NV)), tuple(neg_big for _ in range(NV)))
    sums, mxs = lax.fori_loop(0, nch, chunk, init)
    cnt = (end - start).astype(jnp.float32)
    denom = jnp.maximum(cnt, 1.0)
    for q in range(NV):
      omean[pl.ds(gi * D + q * L, L)] = sums[q] / denom
      omax[pl.ds(gi * D + q * L, L)] = mxs[q]

  pltpu.sync_copy(omean, means.at[pl.ds(w * 2 * D, 2 * D)])
  pltpu.sync_copy(omax, maxs.at[pl.ds(w * 2 * D, 2 * D)])


def _pool_sc(y, batch):
  return pl.kernel(
      _pool_body,
      out_type=(jax.ShapeDtypeStruct((G * D,), jnp.float32),
                jax.ShapeDtypeStruct((G * D,), jnp.float32)),
      mesh=_mesh(),
      scratch_types=[
          pltpu.VMEM((N,), jnp.int32),
          pltpu.VMEM((L, D), jnp.float32),
          pltpu.VMEM((2 * D,), jnp.float32),
          pltpu.VMEM((2 * D,), jnp.float32),
      ],
  )(y, batch)


RB = 256
NB = NP // RB
NSTAT = NB - (NP - N) // RB - 1


def _pre_body(x, w, degb, h, disb):
  dis = 1.0 / jnp.sqrt(degb[...] + 1.0)
  disb[...] = dis
  h[...] = jnp.dot(x[...], w[...], preferred_element_type=jnp.float32) * dis


def _tc_pre(xp, w1, degb):
  return pl.pallas_call(
      _pre_body,
      grid=(NB,),
      in_specs=[
          pl.BlockSpec((RB, D), lambda i: (i, 0)),
          pl.BlockSpec((D, D), lambda i: (0, 0)),
          pl.BlockSpec((RB, D), lambda i: (i, 0)),
      ],
      out_specs=[
          pl.BlockSpec((RB, D), lambda i: (i, 0)),
          pl.BlockSpec((RB, D), lambda i: (i, 0)),
      ],
      out_shape=[
          jax.ShapeDtypeStruct((NP, D), jnp.float32),
          jax.ShapeDtypeStruct((NP, D), jnp.float32),
      ],
  )(xp, w1, degb)


def _bn_body(sparts, hp, disb, gm, bt, wnext, out, acc, *, last):
  p = pl.program_id(0)
  i = pl.program_id(1)
  z = disb[...] * (sparts[0] + sparts[1] + hp[...])
  rows = i * RB + lax.broadcasted_iota(jnp.int32, (RB, D), 0)
  valid = rows < N

  @pl.when(p == 0)
  def _():
    @pl.when(i == 0)
    def _():
      acc[...] = jnp.zeros_like(acc)
    zm = jnp.where(valid, z, 0.0)
    acc[0, :] += jnp.sum(zm, axis=0)

  @pl.when(p == 1)
  def _():
    m = acc[0, :] / N
    zc = jnp.where(valid, z - m[None, :], 0.0)
    acc[1, :] += jnp.sum(zc * zc, axis=0)

  @pl.when(p == 2)
  def _():
    m = acc[0, :] / N
    var = acc[1, :] / N
    y = (z - m[None, :]) / jnp.sqrt(var[None, :] + 1e-5) * gm[...] + bt[...]
    y = jnp.where(valid, jnp.maximum(y, 0.0), 0.0)
    if last:
      out[...] = y
    else:
      out[...] = jnp.dot(y, wnext[...], preferred_element_type=jnp.float32) \
          * disb[...]


def _tc_bn(sparts, hp, disb, gm, bt, wnext):
  last = wnext is None
  body = functools.partial(_bn_body, last=last)
  if last:
    wnext = jnp.zeros((D, D), jnp.float32)
  return pl.pallas_call(
      body,
      grid=(3, NB),
      in_specs=[
          pl.BlockSpec((NC, RB, D), lambda p, i: (0, i, 0)),
          pl.BlockSpec((RB, D), lambda p, i: (i, 0)),
          pl.BlockSpec((RB, D), lambda p, i: (i, 0)),
          pl.BlockSpec((1, D), lambda p, i: (0, 0)),
          pl.BlockSpec((1, D), lambda p, i: (0, 0)),
          pl.BlockSpec((D, D), lambda p, i: (0, 0)),
      ],
      out_specs=pl.BlockSpec((RB, D), lambda p, i: (i, 0)),
      out_shape=jax.ShapeDtypeStruct((NP, D), jnp.float32),
      scratch_shapes=[pltpu.VMEM((2, D), jnp.float32)],
  )(sparts, hp, disb, gm, bt, wnext)


def _heads_body(means, maxs, ws, bs, wd1, bd1, wd2, bd2, wt1, bt1, wt2, bt2,
                out):
  pooled = jnp.concatenate([means[...], maxs[...]], axis=1)
  shared = jnp.maximum(
      jnp.dot(pooled, ws[...], preferred_element_type=jnp.float32) + bs[...],
      0.0)
  hd = jnp.maximum(
      jnp.dot(shared, wd1[...], preferred_element_type=jnp.float32) + bd1[...],
      0.0)
  d = jnp.sum(hd * wd2[...], axis=1) + bd2[0, 0]
  ht = jnp.maximum(
      jnp.dot(shared, wt1[...], preferred_element_type=jnp.float32) + bt1[...],
      0.0)
  t = jnp.sum(ht * wt2[...], axis=1) + bt2[0, 0]
  out[...] = jnp.stack([d, t])


def _tc_heads(means, maxs, ws, bs, wd1, bd1, wd2, bd2, wt1, bt1, wt2, bt2):
  return pl.pallas_call(
      _heads_body,
      out_shape=jax.ShapeDtypeStruct((2, G), jnp.float32),
  )(means, maxs, ws, bs, wd1, bd1, wd2, bd2, wt1, bt1, wt2, bt2)


@jax.jit
def kernel(x, edge_index, batch, W1, b1, g1, be1, W2, b2, g2, be2,
           W3, b3, g3, be3, W4, b4, g4, be4, Ws, bs, Wd1, bd1, Wd2, bd2,
           Wt1, bt1, Wt2, bt2):
  del b1, b2, b3, b4
  pad = jnp.full((EP - E,), N, jnp.int32)
  srcp = jnp.concatenate([edge_index[0], pad])
  dstp = jnp.concatenate([edge_index[1], pad])
  xp = jnp.pad(x, ((0, NP - N), (0, 0)))

  degb = _deg_sc(dstp)
  hp, disb = _tc_pre(xp, W1, degb)
  dst3 = dstp.reshape(32, NCHUNK, CH)

  r2 = lambda a: a.reshape(1, -1)
  for gm, bt, wnext in ((g1, be1, W2), (g2, be2, W3), (g3, be3, W4),
                        (g4, be4, None)):
    sparts = _agg_sc(hp, srcp, dst3)
    hp = _tc_bn(sparts, hp, disb, r2(gm), r2(bt), wnext)

  means, maxs = _pool_sc(hp, batch)
  means = means.reshape(G, D)
  maxs = maxs.reshape(G, D)
  out = _tc_heads(means, maxs, Ws, r2(bs), Wd1, r2(bd1), r2(Wd2),
                  r2(bd2).reshape(1, 1), Wt1, r2(bt1), r2(Wt2),
                  r2(bt2).reshape(1, 1))
  return (out[0], out[1])


# speedup vs baseline: 2.2581x; 2.2581x over previous
import functools
import jax
import jax.numpy as jnp
from jax import lax
from jax.experimental import pallas as pl
from jax.experimental.pallas import tpu as pltpu
from jax.experimental.pallas import tpu_sc as plsc

N = 10000
NP = 10240
NSLAB = 640
E = 320000
CH = 128
EPW = 10240
NCHUNK = EPW // CH
EP = EPW * 32
D = 128
G = 64
NC, NS, L = 2, 16, 16

_mesh = lambda: plsc.VectorSubcoreMesh(
    core_axis_name="c", subcore_axis_name="s", num_cores=NC, num_subcores=NS)


def _zero_vmem_rows(buf, nrows):
  z = jnp.zeros((L,), jnp.float32)
  for r in range(nrows):
    for q in range(D // L):
      buf[r, pl.ds(q * L, L)] = z


def _deg_body(dstp, degb, accd, didx, ones_v, zslab, dslab, rowbuf):
  c = lax.axis_index("c")
  s = lax.axis_index("s")

  @pl.when(c == 0)
  def _():
    z = jnp.zeros((L,), jnp.float32)
    def zb(i, _):
      for q in range(D // L):
        zslab[pl.ds(i * D + q * L, L)] = z
      return 0
    lax.fori_loop(0, NSLAB // D, zb, 0)
    pltpu.sync_copy(zslab, accd.at[pl.ds(s * NSLAB, NSLAB)])
    one = jnp.ones((L,), jnp.float32)
    for q in range(CH // L):
      ones_v[pl.ds(q * L, L)] = one
    plsc.subcore_barrier()

    nch = EP // NS // CH
    def chunk(k, _):
      base = s * (EP // NS) + k * CH
      pltpu.sync_copy(dstp.at[pl.ds(base, CH)], didx)
      pltpu.sync_copy(ones_v, accd.at[didx], add=True)
      return 0
    lax.fori_loop(0, nch, chunk, 0)
    plsc.subcore_barrier()

    pltpu.sync_copy(accd.at[pl.ds(s * NSLAB, NSLAB)], dslab)
    def wb(b, _):
      dvec = dslab[pl.ds(b * L, L)]
      for r in range(L):
        dval = dvec[r]
        v = jnp.full((L,), dval, jnp.float32)
        for q in range(D // L):
          rowbuf[r, pl.ds(q * L, L)] = v
      pltpu.sync_copy(rowbuf, degb.at[pl.ds(s * NSLAB + b * L, L)])
      return 0
    lax.fori_loop(0, NSLAB // L, wb, 0)


def _deg_sc(dstp):
  return pl.kernel(
      _deg_body,
      out_type=jax.ShapeDtypeStruct((NP, D), jnp.float32),
      mesh=_mesh(),
      scratch_types=[
          pltpu.VMEM_SHARED((NP,), jnp.float32),
          pltpu.VMEM((CH,), jnp.int32),
          pltpu.VMEM((CH,), jnp.float32),
          pltpu.VMEM((NSLAB,), jnp.float32),
          pltpu.VMEM((NSLAB,), jnp.float32),
          pltpu.VMEM((L, D), jnp.float32),
      ],
  )(dstp)


def _agg_body(hp, srcp, dst3, out, acc, sidx0, sidx1, didx, rows0, rows1,
              sem0, sem1):
  c = lax.axis_index("c")
  s = lax.axis_index("s")
  wid = c * NS + s
  ebase = wid * EPW

  pltpu.sync_copy(dst3.at[wid], didx)

  _zero_vmem_rows(rows0, CH)
  def zb(i, _):
    pltpu.sync_copy(rows0, acc.at[pl.ds(s * NSLAB + i * CH, CH)])
    return 0
  lax.fori_loop(0, NSLAB // CH, zb, 0)
  plsc.subcore_barrier()

  def fetch(k, sidx, buf, sem):
    pltpu.sync_copy(srcp.at[pl.ds(ebase + k * CH, CH)], sidx)
    pltpu.async_copy(hp.at[sidx], buf, sem)
  fetch(0, sidx0, rows0, sem0)
  fetch(1, sidx1, rows1, sem1)
  def pair(j, _):
    k = j * 2
    pltpu.make_async_copy(hp.at[sidx0], rows0, sem0).wait()
    pltpu.sync_copy(rows0, acc.at[didx.at[k]], add=True)

    @pl.when(j < NCHUNK // 2 - 1)
    def _():
      fetch(k + 2, sidx0, rows0, sem0)
    pltpu.make_async_copy(hp.at[sidx1], rows1, sem1).wait()
    pltpu.sync_copy(rows1, acc.at[didx.at[k + 1]], add=True)

    @pl.when(j < NCHUNK // 2 - 1)
    def _():
      fetch(k + 3, sidx1, rows1, sem1)
    return 0
  lax.fori_loop(0, NCHUNK // 2, pair, 0)
  plsc.subcore_barrier()

  pltpu.sync_copy(acc.at[pl.ds(s * NSLAB, NSLAB)],
                  out.at[c, pl.ds(s * NSLAB, NSLAB)])


def _agg_sc(hp, srcp, dst3):
  return pl.kernel(
      _agg_body,
      out_type=jax.ShapeDtypeStruct((NC, NP, D), jnp.float32),
      mesh=_mesh(),
      scratch_types=[
          pltpu.VMEM_SHARED((NP, D), jnp.float32),
          pltpu.VMEM((CH,), jnp.int32),
          pltpu.VMEM((CH,), jnp.int32),
          pltpu.VMEM((NCHUNK, CH), jnp.int32),
          pltpu.VMEM((CH, D), jnp.float32),
          pltpu.VMEM((CH, D), jnp.float32),
          pltpu.SemaphoreType.DMA,
          pltpu.SemaphoreType.DMA,
      ],
  )(hp, srcp, dst3)


def _pool_body(y, batch, means, maxs, bbuf, rbuf, omean, omax):
  c = lax.axis_index("c")
  s = lax.axis_index("s")
  w = c * NS + s

  pltpu.sync_copy(batch, bbuf)

  def lower_bound(gval):
    one = jnp.ones((L,), jnp.int32)
    zero = jnp.zeros((L,), jnp.int32)
    def sb(i, acc):
      v = bbuf[pl.ds(i * L, L)]
      return acc + jnp.minimum(jnp.maximum(gval - v, zero), one)
    acc = lax.fori_loop(0, N // L, sb, jnp.zeros((L,), jnp.int32))
    tot = acc[0]
    for j in range(1, L):
      tot = tot + acc[j]
    return tot

  g0 = w * 2
  lb0 = lower_bound(g0)
  lb1 = lower_bound(g0 + 1)
  lb2 = lower_bound(g0 + 2)

  NV = D // L
  BIG = jnp.float32(3.0e38)
  neg_big = jnp.full((L,), -BIG, jnp.float32)
  zeros = jnp.zeros((L,), jnp.float32)

  for gi, (start, end) in enumerate([(lb0, lb1), (lb1, lb2)]):
    astart = (start // L) * L
    nch = (end - astart + L - 1) // L
    def chunk(k, carry):
      sums, mxs = carry
      base = astart + k * L
      pltpu.sync_copy(y.at[pl.ds(base, L)], rbuf)
      for r in range(L):
        rg = base + r
        valid = jnp.logical_and(rg >= start, rg < end)
        vf = jnp.where(valid, jnp.float32(1.0), jnp.float32(0.0))
        vm = jnp.full((L,), vf, jnp.float32)
        vbias = jnp.full((L,), (vf - 1.0) * BIG, jnp.float32)
        for q in range(NV):
          v = rbuf[r, pl.ds(q * L, L)]
          sums = sums[:q] + (sums[q] + v * vm,) + sums[q + 1:]
          mxs = mxs[:q] + (jnp.maximum(mxs[q], v * vm + vbias),) + mxs[q + 1:]
      return (sums, mxs)
    init = (tuple(zeros for _ in range(NV)), tuple(neg_big for _ in range(NV)))
    sums, mxs = lax.fori_loop(0, nch, chunk, init)
    cnt = (end - start).astype(jnp.float32)
    denom = jnp.maximum(cnt, 1.0)
    for q in range(NV):
      omean[pl.ds(gi * D + q * L, L)] = sums[q] / denom
      omax[pl.ds(gi * D + q * L, L)] = mxs[q]

  pltpu.sync_copy(omean, means.at[pl.ds(w * 2 * D, 2 * D)])
  pltpu.sync_copy(omax, maxs.at[pl.ds(w * 2 * D, 2 * D)])


def _pool_sc(y, batch):
  return pl.kernel(
      _pool_body,
      out_type=(jax.ShapeDtypeStruct((G * D,), jnp.float32),
                jax.ShapeDtypeStruct((G * D,), jnp.float32)),
      mesh=_mesh(),
      scratch_types=[
          pltpu.VMEM((N,), jnp.int32),
          pltpu.VMEM((L, D), jnp.float32),
          pltpu.VMEM((2 * D,), jnp.float32),
          pltpu.VMEM((2 * D,), jnp.float32),
      ],
  )(y, batch)


RB = 256
NB = NP // RB
NSTAT = NB - (NP - N) // RB - 1


def _pre_body(x, w, degb, h, disb):
  dis = 1.0 / jnp.sqrt(degb[...] + 1.0)
  disb[...] = dis
  h[...] = jnp.dot(x[...], w[...], preferred_element_type=jnp.float32) * dis


def _tc_pre(xp, w1, degb):
  return pl.pallas_call(
      _pre_body,
      grid=(NB,),
      in_specs=[
          pl.BlockSpec((RB, D), lambda i: (i, 0)),
          pl.BlockSpec((D, D), lambda i: (0, 0)),
          pl.BlockSpec((RB, D), lambda i: (i, 0)),
      ],
      out_specs=[
          pl.BlockSpec((RB, D), lambda i: (i, 0)),
          pl.BlockSpec((RB, D), lambda i: (i, 0)),
      ],
      out_shape=[
          jax.ShapeDtypeStruct((NP, D), jnp.float32),
          jax.ShapeDtypeStruct((NP, D), jnp.float32),
      ],
  )(xp, w1, degb)


def _bn_body(sparts, hp, disb, gm, bt, wnext, out, acc, *, last):
  p = pl.program_id(0)
  i = pl.program_id(1)
  z = disb[...] * (sparts[0] + sparts[1] + hp[...])
  rows = i * RB + lax.broadcasted_iota(jnp.int32, (RB, D), 0)
  valid = rows < N

  @pl.when(p == 0)
  def _():
    @pl.when(i == 0)
    def _():
      acc[...] = jnp.zeros_like(acc)
    zm = jnp.where(valid, z, 0.0)
    acc[0, :] += jnp.sum(zm, axis=0)

  @pl.when(p == 1)
  def _():
    m = acc[0, :] / N
    zc = jnp.where(valid, z - m[None, :], 0.0)
    acc[1, :] += jnp.sum(zc * zc, axis=0)

  @pl.when(p == 2)
  def _():
    m = acc[0, :] / N
    var = acc[1, :] / N
    y = (z - m[None, :]) / jnp.sqrt(var[None, :] + 1e-5) * gm[...] + bt[...]
    y = jnp.where(valid, jnp.maximum(y, 0.0), 0.0)
    if last:
      out[...] = y
    else:
      out[...] = jnp.dot(y, wnext[...], preferred_element_type=jnp.float32) \
          * disb[...]


def _tc_bn(sparts, hp, disb, gm, bt, wnext):
  last = wnext is None
  body = functools.partial(_bn_body, last=last)
  if last:
    wnext = jnp.zeros((D, D), jnp.float32)
  return pl.pallas_call(
      body,
      grid=(3, NB),
      in_specs=[
          pl.BlockSpec((NC, RB, D), lambda p, i: (0, i, 0)),
          pl.BlockSpec((RB, D), lambda p, i: (i, 0)),
          pl.BlockSpec((RB, D), lambda p, i: (i, 0)),
          pl.BlockSpec((1, D), lambda p, i: (0, 0)),
          pl.BlockSpec((1, D), lambda p, i: (0, 0)),
          pl.BlockSpec((D, D), lambda p, i: (0, 0)),
      ],
      out_specs=pl.BlockSpec((RB, D), lambda p, i: (i, 0)),
      out_shape=jax.ShapeDtypeStruct((NP, D), jnp.float32),
      scratch_shapes=[pltpu.VMEM((2, D), jnp.float32)],
  )(sparts, hp, disb, gm, bt, wnext)


def _heads_body(means, maxs, ws, bs, wd1, bd1, wd2, bd2, wt1, bt1, wt2, bt2,
                out):
  pooled = jnp.concatenate([means[...], maxs[...]], axis=1)
  shared = jnp.maximum(
      jnp.dot(pooled, ws[...], preferred_element_type=jnp.float32) + bs[...],
      0.0)
  hd = jnp.maximum(
      jnp.dot(shared, wd1[...], preferred_element_type=jnp.float32) + bd1[...],
      0.0)
  d = jnp.sum(hd * wd2[...], axis=1) + bd2[0, 0]
  ht = jnp.maximum(
      jnp.dot(shared, wt1[...], preferred_element_type=jnp.float32) + bt1[...],
      0.0)
  t = jnp.sum(ht * wt2[...], axis=1) + bt2[0, 0]
  out[...] = jnp.stack([d, t])


def _tc_heads(means, maxs, ws, bs, wd1, bd1, wd2, bd2, wt1, bt1, wt2, bt2):
  return pl.pallas_call(
      _heads_body,
      out_shape=jax.ShapeDtypeStruct((2, G), jnp.float32),
  )(means, maxs, ws, bs, wd1, bd1, wd2, bd2, wt1, bt1, wt2, bt2)


@jax.jit
def kernel(x, edge_index, batch, W1, b1, g1, be1, W2, b2, g2, be2,
           W3, b3, g3, be3, W4, b4, g4, be4, Ws, bs, Wd1, bd1, Wd2, bd2,
           Wt1, bt1, Wt2, bt2):
  del b1, b2, b3, b4
  pad = N + (jnp.arange(EP - E, dtype=jnp.int32) % (NP - N))
  srcp = jnp.concatenate([edge_index[0], pad])
  dstp = jnp.concatenate([edge_index[1], pad])
  xp = jnp.pad(x, ((0, NP - N), (0, 0)))

  degb = _deg_sc(dstp)
  hp, disb = _tc_pre(xp, W1, degb)
  dst3 = dstp.reshape(32, NCHUNK, CH)

  r2 = lambda a: a.reshape(1, -1)
  for gm, bt, wnext in ((g1, be1, W2), (g2, be2, W3), (g3, be3, W4),
                        (g4, be4, None)):
    sparts = _agg_sc(hp, srcp, dst3)
    hp = _tc_bn(sparts, hp, disb, r2(gm), r2(bt), wnext)

  means, maxs = _pool_sc(hp, batch)
  means = means.reshape(G, D)
  maxs = maxs.reshape(G, D)
  out = _tc_heads(means, maxs, Ws, r2(bs), Wd1, r2(bd1), r2(Wd2),
                  r2(bd2).reshape(1, 1), Wt1, r2(bt1), r2(Wt2),
                  r2(bt2).reshape(1, 1))
  return (out[0], out[1])
